# baseline (speedup 1.0000x reference)
import functools

import jax
import jax.numpy as jnp
from jax import lax
from jax.experimental import pallas as pl
from jax.experimental.pallas import tpu as pltpu
from jax.experimental.pallas import tpu_sc as plsc

N_NODES = 50000
N_EDGES = 1600000
NPAD = 50176
SLICE = NPAD // 16

KE = 14.3996454784255
A0 = 0.52917721092
ZBL_A = (0.1818, 0.5099, 0.2802, 0.02817)
ZBL_B = (3.2, 0.9423, 0.4029, 0.2016)
INV_A = 1.0 / (0.88534 * A0)

ROW = 80
ROWS_PER_CHUNK = 25
CHUNKS = 25
ROWS_PER_TILE = N_EDGES // 32 // ROW
GROUP = 5


def _znode_body(attrs_ref, an_ref, z_ref, zp_ref):
    a = attrs_ref[...]
    w = an_ref[...]
    aw = jax.lax.dot_general(a, w, (((1,), (0,)), ((), ())),
                             preferred_element_type=jnp.float32)
    z = jnp.max(aw, axis=1)
    zp = jnp.where(z > 0.0, jnp.exp(0.23 * jnp.log(z)), 0.0)
    z_ref[...] = z * (0.25 * KE) ** 0.5
    zp_ref[...] = zp * INV_A


def _combine_body(p_ref, o_ref):
    o_ref[...] = p_ref[0, :] + p_ref[1, :]


def _edge_body(z_hbm, zp_hbm, eidx_hbm, len_hbm, rv_hbm, out_hbm,
               ztab, zptab, src2, dst2, len2, vh, rv, zbuf, acc,
               insem0, insem1, ssem0, ssem1):
    insems = (insem0, insem1)
    ssems = (ssem0, ssem1)
    c = lax.axis_index("c")
    s = lax.axis_index("s")

    pltpu.sync_copy(z_hbm, ztab)
    pltpu.sync_copy(zp_hbm, zptab)
    pltpu.sync_copy(rv_hbm, rv)

    zero16 = jnp.zeros((16,), jnp.float32)

    def _zb(i, carry):
        zbuf[pl.ds(i * 16, 16)] = zero16
        return carry

    lax.fori_loop(0, SLICE // 16, _zb, 0)
    pltpu.sync_copy(zbuf, acc.at[pl.ds(s * SLICE, SLICE)])
    plsc.subcore_barrier()

    inv_rmax = rv[...]

    base_row = c * (ROWS_PER_TILE * 16) + s * ROWS_PER_TILE

    def _in_descs(ch, p):
        rb = base_row + ch * ROWS_PER_CHUNK
        return (
            pltpu.make_async_copy(
                eidx_hbm.at[0, pl.ds(rb, ROWS_PER_CHUNK)], src2.at[p],
                insems[p]),
            pltpu.make_async_copy(
                eidx_hbm.at[1, pl.ds(rb, ROWS_PER_CHUNK)], dst2.at[p],
                insems[p]),
            pltpu.make_async_copy(
                len_hbm.at[pl.ds(rb * ROW, ROWS_PER_CHUNK * ROW)], len2.at[p],
                insems[p]),
        )

    def _start_in(ch, p):
        for d in _in_descs(ch, p):
            d.start()

    def _wait_in(ch, p):
        for d in _in_descs(ch, p):
            d.wait()

    def _process(ch, p):
        _wait_in(ch, p)

        def _row(j, carry):
            for k in range(ROW // 16):
                sl = pl.ds(k * 16, 16)
                isrc = src2[p, j, sl]
                idst = dst2[p, j, sl]
                r = jnp.maximum(len2[p, pl.ds(j * ROW + k * 16, 16)], 0.2)
                zi = plsc.load_gather(ztab, [isrc])
                zj = plsc.load_gather(ztab, [idst])
                pi = plsc.load_gather(zptab, [isrc])
                pj = plsc.load_gather(zptab, [idst])
                x = r * (pi + pj)
                phi = (ZBL_A[0] * jnp.exp(-ZBL_B[0] * x)
                       + ZBL_A[1] * jnp.exp(-ZBL_B[1] * x)
                       + ZBL_A[2] * jnp.exp(-ZBL_B[2] * x)
                       + ZBL_A[3] * jnp.exp(-ZBL_B[3] * x))
                xc = jnp.minimum(r * inv_rmax, 1.0)
                om = 1.0 - xc
                om2 = om * om
                cut = om2 * om2 * om2
                vh[p, j, sl] = zi * zj * phi * cut / r
            pltpu.make_async_copy(
                vh.at[p, j], acc.at[src2.at[p, j]], ssems[p]).start(add=True)
            pltpu.make_async_copy(
                vh.at[p, j], acc.at[dst2.at[p, j]], ssems[p]).start(add=True)
            return carry

        lax.fori_loop(0, ROWS_PER_CHUNK, _row, 0)

        def _drain(j, carry):
            pltpu.make_async_copy(
                vh.at[p, j], acc.at[src2.at[p, j]], ssems[p]).wait()
            pltpu.make_async_copy(
                vh.at[p, j], acc.at[dst2.at[p, j]], ssems[p]).wait()
            return carry

        lax.fori_loop(0, ROWS_PER_CHUNK, _drain, 0)

    _start_in(0, 0)

    def _pair(g, carry):
        ch0 = 2 * g
        _start_in(ch0 + 1, 1)
        _process(ch0, 0)
        _start_in(ch0 + 2, 0)
        _process(ch0 + 1, 1)
        return carry

    lax.fori_loop(0, CHUNKS // 2, _pair, 0)
    _process(CHUNKS - 1, 0)

    plsc.subcore_barrier()

    pltpu.sync_copy(acc.at[pl.ds(s * SLICE, SLICE)],
                    out_hbm.at[c, pl.ds(s * SLICE, SLICE)])


def kernel(lengths, node_attrs, edge_index, atomic_numbers, r_max):
    f32 = jnp.float32

    an = jnp.broadcast_to(atomic_numbers[:, None], (10, 128))
    blk = NPAD // 7
    z_node, zp_node = pl.pallas_call(
        _znode_body,
        grid=(7,),
        in_specs=[
            pl.BlockSpec((blk, 10), lambda i: (i, 0)),
            pl.BlockSpec((10, 128), lambda i: (0, 0)),
        ],
        out_specs=[
            pl.BlockSpec((blk,), lambda i: (i,)),
            pl.BlockSpec((blk,), lambda i: (i,)),
        ],
        out_shape=[
            jax.ShapeDtypeStruct((NPAD,), f32),
            jax.ShapeDtypeStruct((NPAD,), f32),
        ],
    )(node_attrs, an)

    eidx3 = edge_index.astype(jnp.int32).reshape(2, N_EDGES // ROW, ROW)
    rvec = jnp.broadcast_to((1.0 / r_max).astype(f32), (16,))

    mesh = plsc.VectorSubcoreMesh(core_axis_name="c", subcore_axis_name="s")
    partial = pl.kernel(
        _edge_body,
        out_type=jax.ShapeDtypeStruct((2, NPAD), f32),
        mesh=mesh,
        compiler_params=pltpu.CompilerParams(
            use_tc_tiling_on_sc=False, needs_layout_passes=False),
        scratch_types=[
            pltpu.VMEM((NPAD,), f32),
            pltpu.VMEM((NPAD,), f32),
            pltpu.VMEM((2, ROWS_PER_CHUNK, ROW), jnp.int32),
            pltpu.VMEM((2, ROWS_PER_CHUNK, ROW), jnp.int32),
            pltpu.VMEM((2, ROWS_PER_CHUNK * ROW), f32),
            pltpu.VMEM((2, ROWS_PER_CHUNK, ROW), f32),
            pltpu.VMEM((16,), f32),
            pltpu.VMEM((SLICE,), f32),
            pltpu.VMEM_SHARED((NPAD,), f32),
            pltpu.SemaphoreType.DMA,
            pltpu.SemaphoreType.DMA,
            pltpu.SemaphoreType.DMA,
            pltpu.SemaphoreType.DMA,
        ],
    )(z_node, zp_node, eidx3, lengths, rvec)

    node_e = pl.pallas_call(
        _combine_body,
        out_shape=jax.ShapeDtypeStruct((NPAD,), f32),
    )(partial)
    return node_e[:N_NODES]


# speedup vs baseline: 1.1786x; 1.1786x over previous
import functools

import jax
import jax.numpy as jnp
from jax import lax
from jax.experimental import pallas as pl
from jax.experimental.pallas import tpu as pltpu
from jax.experimental.pallas import tpu_sc as plsc

N_NODES = 50000
N_EDGES = 1600000
NPAD = 50176
SLICE = NPAD // 16

KE = 14.3996454784255
A0 = 0.52917721092
ZBL_A = (0.1818, 0.5099, 0.2802, 0.02817)
ZBL_B = (3.2, 0.9423, 0.4029, 0.2016)
INV_A = 1.0 / (0.88534 * A0)

ROW = 80
ROWS_PER_CHUNK = 25
CHUNKS = 25
ROWS_PER_TILE = N_EDGES // 32 // ROW
GROUP = 5


def _znode_body(attrs_ref, an_ref, z_ref, zp_ref):
    a = attrs_ref[...]
    w = an_ref[0:1, :]
    z = jnp.sum(a * w, axis=1)
    zp = jnp.where(z > 0.0, jnp.exp(0.23 * jnp.log(z)), 0.0)
    z_ref[...] = z * (0.25 * KE) ** 0.5
    zp_ref[...] = zp * INV_A


def _combine_body(p_ref, o_ref):
    o_ref[...] = p_ref[0, :] + p_ref[1, :]


def _edge_body(z_hbm, zp_hbm, eidx_hbm, len_hbm, rv_hbm, out_hbm,
               ztab, zptab, src2, dst2, len2, vh, rv, zbuf, acc,
               insem0, insem1, ssem0, ssem1):
    insems = (insem0, insem1)
    ssems = (ssem0, ssem1)
    c = lax.axis_index("c")
    s = lax.axis_index("s")

    pltpu.sync_copy(z_hbm, ztab)
    pltpu.sync_copy(zp_hbm, zptab)
    pltpu.sync_copy(rv_hbm, rv)

    zero16 = jnp.zeros((16,), jnp.float32)

    def _zb(i, carry):
        zbuf[pl.ds(i * 16, 16)] = zero16
        return carry

    lax.fori_loop(0, SLICE // 16, _zb, 0)
    pltpu.sync_copy(zbuf, acc.at[pl.ds(s * SLICE, SLICE)])
    plsc.subcore_barrier()

    inv_rmax = rv[...]

    base_row = c * (ROWS_PER_TILE * 16) + s * ROWS_PER_TILE

    def _in_descs(ch, p):
        rb = base_row + ch * ROWS_PER_CHUNK
        return (
            pltpu.make_async_copy(
                eidx_hbm.at[0, pl.ds(rb, ROWS_PER_CHUNK)], src2.at[p],
                insems[p]),
            pltpu.make_async_copy(
                eidx_hbm.at[1, pl.ds(rb, ROWS_PER_CHUNK)], dst2.at[p],
                insems[p]),
            pltpu.make_async_copy(
                len_hbm.at[pl.ds(rb * ROW, ROWS_PER_CHUNK * ROW)], len2.at[p],
                insems[p]),
        )

    def _start_in(ch, p):
        for d in _in_descs(ch, p):
            d.start()

    def _wait_in(ch, p):
        for d in _in_descs(ch, p):
            d.wait()

    def _process(ch, p):
        _wait_in(ch, p)

        @plsc.parallel_loop(0, ROWS_PER_CHUNK, 1, unroll=2)
        def _row(j):
            for k in range(ROW // 16):
                sl = pl.ds(k * 16, 16)
                isrc = src2[p, j, sl]
                idst = dst2[p, j, sl]
                r = jnp.maximum(len2[p, pl.ds(j * ROW + k * 16, 16)], 0.2)
                zi = plsc.load_gather(ztab, [isrc])
                zj = plsc.load_gather(ztab, [idst])
                pi = plsc.load_gather(zptab, [isrc])
                pj = plsc.load_gather(zptab, [idst])
                x = r * (pi + pj)
                phi = (ZBL_A[0] * jnp.exp(-ZBL_B[0] * x)
                       + ZBL_A[1] * jnp.exp(-ZBL_B[1] * x)
                       + ZBL_A[2] * jnp.exp(-ZBL_B[2] * x)
                       + ZBL_A[3] * jnp.exp(-ZBL_B[3] * x))
                xc = jnp.minimum(r * inv_rmax, 1.0)
                om = 1.0 - xc
                om2 = om * om
                cut = om2 * om2 * om2
                vh[p, j, sl] = zi * zj * phi * cut / r
            pltpu.make_async_copy(
                vh.at[p, j], acc.at[src2.at[p, j]], ssems[p]).start(add=True)
            pltpu.make_async_copy(
                vh.at[p, j], acc.at[dst2.at[p, j]], ssems[p]).start(add=True)

        def _drain(j, carry):
            pltpu.make_async_copy(
                vh.at[p, j], acc.at[src2.at[p, j]], ssems[p]).wait()
            pltpu.make_async_copy(
                vh.at[p, j], acc.at[dst2.at[p, j]], ssems[p]).wait()
            return carry

        lax.fori_loop(0, ROWS_PER_CHUNK, _drain, 0)

    _start_in(0, 0)

    def _pair(g, carry):
        ch0 = 2 * g
        _start_in(ch0 + 1, 1)
        _process(ch0, 0)
        _start_in(ch0 + 2, 0)
        _process(ch0 + 1, 1)
        return carry

    lax.fori_loop(0, CHUNKS // 2, _pair, 0)
    _process(CHUNKS - 1, 0)

    plsc.subcore_barrier()

    pltpu.sync_copy(acc.at[pl.ds(s * SLICE, SLICE)],
                    out_hbm.at[c, pl.ds(s * SLICE, SLICE)])


def kernel(lengths, node_attrs, edge_index, atomic_numbers, r_max):
    f32 = jnp.float32

    an = jnp.broadcast_to(atomic_numbers[None, :], (8, 10))
    blk = NPAD // 7
    z_node, zp_node = pl.pallas_call(
        _znode_body,
        grid=(7,),
        in_specs=[
            pl.BlockSpec((blk, 10), lambda i: (i, 0)),
            pl.BlockSpec((8, 10), lambda i: (0, 0)),
        ],
        out_specs=[
            pl.BlockSpec((blk,), lambda i: (i,)),
            pl.BlockSpec((blk,), lambda i: (i,)),
        ],
        out_shape=[
            jax.ShapeDtypeStruct((NPAD,), f32),
            jax.ShapeDtypeStruct((NPAD,), f32),
        ],
    )(node_attrs, an)

    eidx3 = edge_index.astype(jnp.int32).reshape(2, N_EDGES // ROW, ROW)
    rvec = jnp.broadcast_to((1.0 / r_max).astype(f32), (16,))

    mesh = plsc.VectorSubcoreMesh(core_axis_name="c", subcore_axis_name="s")
    partial = pl.kernel(
        _edge_body,
        out_type=jax.ShapeDtypeStruct((2, NPAD), f32),
        mesh=mesh,
        compiler_params=pltpu.CompilerParams(
            use_tc_tiling_on_sc=False, needs_layout_passes=False),
        scratch_types=[
            pltpu.VMEM((NPAD,), f32),
            pltpu.VMEM((NPAD,), f32),
            pltpu.VMEM((2, ROWS_PER_CHUNK, ROW), jnp.int32),
            pltpu.VMEM((2, ROWS_PER_CHUNK, ROW), jnp.int32),
            pltpu.VMEM((2, ROWS_PER_CHUNK * ROW), f32),
            pltpu.VMEM((2, ROWS_PER_CHUNK, ROW), f32),
            pltpu.VMEM((16,), f32),
            pltpu.VMEM((SLICE,), f32),
            pltpu.VMEM_SHARED((NPAD,), f32),
            pltpu.SemaphoreType.DMA,
            pltpu.SemaphoreType.DMA,
            pltpu.SemaphoreType.DMA,
            pltpu.SemaphoreType.DMA,
        ],
    )(z_node, zp_node, eidx3, lengths, rvec)

    node_e = pl.pallas_call(
        _combine_body,
        out_shape=jax.ShapeDtypeStruct((NPAD,), f32),
    )(partial)
    return node_e[:N_NODES]
